# Initial kernel scaffold; baseline (speedup 1.0000x reference)
import jax
import jax.numpy as jnp
from jax.experimental import pallas as pl


def kernel(updates, mask):
    raise NotImplementedError("write your pallas kernel here")


# speedup vs baseline: 1.2505x; 1.2505x over previous
import functools

import jax
import jax.numpy as jnp
from jax import lax
from jax.experimental import pallas as pl
from jax.experimental.pallas import tpu as pltpu
from jax.experimental.pallas import tpu_sc as plsc

B, H, W_IN, C = 2, 256, 256, 96
FLAT = (2 * H) * (2 * W_IN) * C
N = B * H * W_IN * C
OUT = B * FLAT

NC, NS = 2, 16
WIN = 1 << 20
NWIN = OUT // WIN
NPASS = NWIN // NC
SH = WIN + 2048
ZSLICE = SH // NS
BLK = 2048
PER_TILE = N // NS
NBLK = PER_TILE // BLK
FSL = WIN // NS


def _sc_body(idx_hbm, val_hbm, out_hbm,
             idx_buf, val_buf, cbuf_off, cbuf_val, offs128, vals128,
             zero_buf, shared):
  c = lax.axis_index("c")
  s = lax.axis_index("s")
  lane = lax.iota(jnp.int32, 16)
  zero16f = jnp.zeros((16,), jnp.float32)
  trash_vec = (WIN + s * 128).astype(jnp.int32) + lane

  def zb(i, _):
    zero_buf[pl.ds(16 * i, 16)] = zero16f
    return 0
  lax.fori_loop(0, zero_buf.shape[0] // 16, zb, 0)

  batch_off = jnp.where(s < 8, 0, FLAT).astype(jnp.int32)

  def one_pass(p, _):
    w = NC * p + c
    win_base = (w * WIN).astype(jnp.int32)
    base_const = win_base - batch_off

    zb0 = s * ZSLICE
    for j in range(ZSLICE // 8192):
      pltpu.sync_copy(zero_buf, shared.at[pl.ds(zb0 + 8192 * j, 8192)])
    rem = ZSLICE % 8192
    if rem:
      pltpu.sync_copy(zero_buf.at[pl.ds(0, rem)],
                      shared.at[pl.ds(zb0 + (ZSLICE - rem), rem)])
    plsc.subcore_barrier()

    def block(b, _):
      g0 = s * PER_TILE + b * BLK
      pltpu.sync_copy(idx_hbm.at[pl.ds(g0, BLK)], idx_buf)
      pltpu.sync_copy(val_hbm.at[pl.ds(g0, BLK)], val_buf)

      def vreg(i, cnt):
        iv = idx_buf[pl.ds(16 * i, 16)]
        uv = val_buf[pl.ds(16 * i, 16)]
        off = iv - base_const
        m = (off >= 0) & (off < WIN)
        mi = m.astype(jnp.int32)
        pos = cnt + plsc.cumsum(mi) - 1
        plsc.store_scatter(cbuf_off, [pos], off, mask=m)
        plsc.store_scatter(cbuf_val, [pos], uv, mask=m)
        return cnt + jnp.sum(mi)

      cnt = lax.fori_loop(0, BLK // 16, vreg, jnp.int32(0))

      for j in range(8):
        cbuf_off[pl.ds(cnt + 16 * j, 16)] = trash_vec
        cbuf_val[pl.ds(cnt + 16 * j, 16)] = zero16f

      ndma = (cnt + 127) // 128

      def fire(j, _):
        for k in range(8):
          offs128[pl.ds(16 * k, 16)] = cbuf_off[pl.ds(128 * j + 16 * k, 16)]
          vals128[pl.ds(16 * k, 16)] = cbuf_val[pl.ds(128 * j + 16 * k, 16)]
        pltpu.sync_copy(vals128, shared.at[offs128], add=True)
        return 0

      lax.fori_loop(0, ndma, fire, 0)
      return 0

    lax.fori_loop(0, NBLK, block, 0)
    plsc.subcore_barrier()

    fsrc = s * FSL
    fdst = win_base + s * FSL
    for j in range(FSL // 8192):
      pltpu.sync_copy(shared.at[pl.ds(fsrc + 8192 * j, 8192)],
                      out_hbm.at[pl.ds(fdst + 8192 * j, 8192)])
    plsc.subcore_barrier()
    return 0

  lax.fori_loop(0, NPASS, one_pass, 0)


_scatter_add = pl.kernel(
    _sc_body,
    out_type=jax.ShapeDtypeStruct((OUT,), jnp.float32),
    mesh=plsc.VectorSubcoreMesh(core_axis_name="c", subcore_axis_name="s",
                                num_cores=NC, num_subcores=NS),
    compiler_params=pltpu.CompilerParams(needs_layout_passes=False),
    scratch_types=[
        pltpu.VMEM((BLK,), jnp.int32),
        pltpu.VMEM((BLK,), jnp.float32),
        pltpu.VMEM((BLK + 256,), jnp.int32),
        pltpu.VMEM((BLK + 256,), jnp.float32),
        pltpu.VMEM((128,), jnp.int32),
        pltpu.VMEM((128,), jnp.float32),
        pltpu.VMEM((8192,), jnp.float32),
        pltpu.VMEM_SHARED((SH,), jnp.float32),
    ],
)


@jax.jit
def kernel(updates, mask):
  flat_updates = updates.reshape(-1)
  flat_mask = mask.reshape(-1).astype(jnp.int32)
  out = _scatter_add(flat_mask, flat_updates)
  return out.reshape(B, 2 * H, 2 * W_IN, C)


# speedup vs baseline: 8.8183x; 7.0520x over previous
import jax
import jax.numpy as jnp
from jax import lax
from jax.experimental import pallas as pl
from jax.experimental.pallas import tpu as pltpu
from jax.experimental.pallas import tpu_sc as plsc

B, H, W_IN, C = 2, 256, 256, 96
FLAT = (2 * H) * (2 * W_IN) * C
N = B * H * W_IN * C
HALF = N // 2
OUT = B * FLAT

NC, NS = 2, 16
WIN = 1802240
NWIN = -(-OUT // WIN)
NPASS = NWIN // NC
SH = WIN + 2048
ZSLICE = SH // NS
FSL = WIN // NS
LAST_WIN = OUT - (NWIN - 1) * WIN
LAST_FSL = LAST_WIN // NS
BLK = 2048
CROWS = (BLK + 256) // 16


def _sc_body(idx_hbm, val_hbm, zeros_hbm, out_hbm,
             idx_buf0, idx_buf1, val_buf0, val_buf1,
             coff0, coff1, cval0, cval1,
             shared, sem0, sem1, fsem0, fsem1):
  idx_bufs = (idx_buf0, idx_buf1)
  val_bufs = (val_buf0, val_buf1)
  coffs = (coff0, coff1)
  cvals = (cval0, cval1)
  sems = (sem0, sem1)
  fsems = (fsem0, fsem1)
  c = lax.axis_index("c")
  s = lax.axis_index("s")
  lane = lax.iota(jnp.int32, 16)
  zero16f = jnp.zeros((16,), jnp.float32)
  zero16i = jnp.zeros((16,), jnp.int32)
  trash_vec = (WIN + s * 128).astype(jnp.int32) + lane

  def fire_desc(par, j):
    return pltpu.make_async_copy(
        cvals[par].at[pl.ds(128 * j, 128)],
        shared.at[coffs[par].at[pl.ds(128 * j, 128)]],
        fsems[par])

  def drain(par, pend):
    def one(j, _):
      fire_desc(par, j).wait()
      return 0
    lax.fori_loop(0, pend, one, 0)

  def one_pass(p, _):
    w = NC * p + c
    win_base = w * WIN
    straddle = (win_base < FLAT) & (win_base + WIN > FLAT)
    per_tile = jnp.where(straddle, N // NS, HALF // NS)
    scan_lo = jnp.where(straddle | (win_base < FLAT), 0, HALF)
    nblk = per_tile // BLK
    tile_lo = scan_lo + s * per_tile
    batch_off = jnp.where(tile_lo >= HALF, FLAT, 0)
    base_const = win_base - batch_off

    pltpu.sync_copy(zeros_hbm.at[pl.ds(s * ZSLICE, ZSLICE)],
                    shared.at[pl.ds(s * ZSLICE, ZSLICE)])
    plsc.subcore_barrier()

    def copies(b, par):
      g0 = tile_lo + b * BLK
      return (
          pltpu.make_async_copy(idx_hbm.at[pl.ds(g0, BLK)],
                                idx_bufs[par], sems[par]),
          pltpu.make_async_copy(val_hbm.at[pl.ds(g0, BLK)],
                                val_bufs[par], sems[par]),
      )

    def issue(b, par):
      for cp in copies(b, par):
        cp.start()

    issue(0, 0)

    def process(par):
      ib = idx_bufs[par]
      vb = val_bufs[par]
      cbuf_off = coffs[par]
      cbuf_val = cvals[par]

      def vreg(i, cntv):
        iv = ib[pl.ds(16 * i, 16)]
        uv = vb[pl.ds(16 * i, 16)]
        off = iv - base_const
        m = (off >= 0) & (off < WIN)
        pos = (cntv << 4) + lane
        plsc.store_scatter(cbuf_off, [pos], off, mask=m)
        plsc.store_scatter(cbuf_val, [pos], uv, mask=m)
        return cntv + m.astype(jnp.int32)

      cntv = lax.fori_loop(0, BLK // 16, vreg, zero16i)

      maxc = jnp.max(cntv)
      minc = jnp.min(cntv)
      mpad = ((maxc + 7) // 8) * 8

      def fill(r, _):
        fm = r >= cntv
        rowpos = (r << 4) + lane
        plsc.store_scatter(cbuf_off, [rowpos], trash_vec, mask=fm)
        plsc.store_scatter(cbuf_val, [rowpos], zero16f, mask=fm)
        return 0

      lax.fori_loop(minc, mpad, fill, 0)

      ndma = mpad // 8

      def fire(j, _):
        fire_desc(par, j).start(add=True)
        return 0

      lax.fori_loop(0, ndma, fire, 0)
      return ndma

    def pair(i, pend):
      p0, p1 = pend
      for par in range(2):
        b = 2 * i + par
        for cp in copies(b, par):
          cp.wait()

        @pl.when(b + 1 < nblk)
        def _():
          issue(b + 1, 1 - par)

        drain(par, p0 if par == 0 else p1)
        nd = process(par)
        if par == 0:
          p0 = nd
        else:
          p1 = nd
      return (p0, p1)

    p0, p1 = lax.fori_loop(0, nblk // 2, pair,
                           (jnp.int32(0), jnp.int32(0)))
    drain(0, p0)
    drain(1, p1)
    plsc.subcore_barrier()

    @pl.when(w < NWIN - 1)
    def _():
      pltpu.sync_copy(shared.at[pl.ds(s * FSL, FSL)],
                      out_hbm.at[pl.ds(win_base + s * FSL, FSL)])

    @pl.when(w == NWIN - 1)
    def _():
      pltpu.sync_copy(shared.at[pl.ds(s * LAST_FSL, LAST_FSL)],
                      out_hbm.at[pl.ds(win_base + s * LAST_FSL, LAST_FSL)])

    plsc.subcore_barrier()
    return 0

  lax.fori_loop(0, NPASS, one_pass, 0)


_scatter_add = pl.kernel(
    _sc_body,
    out_type=jax.ShapeDtypeStruct((OUT,), jnp.float32),
    mesh=plsc.VectorSubcoreMesh(core_axis_name="c", subcore_axis_name="s",
                                num_cores=NC, num_subcores=NS),
    compiler_params=pltpu.CompilerParams(needs_layout_passes=False),
    scratch_types=[
        pltpu.VMEM((BLK,), jnp.int32),
        pltpu.VMEM((BLK,), jnp.int32),
        pltpu.VMEM((BLK,), jnp.float32),
        pltpu.VMEM((BLK,), jnp.float32),
        pltpu.VMEM((CROWS * 16,), jnp.int32),
        pltpu.VMEM((CROWS * 16,), jnp.int32),
        pltpu.VMEM((CROWS * 16,), jnp.float32),
        pltpu.VMEM((CROWS * 16,), jnp.float32),
        pltpu.VMEM_SHARED((SH,), jnp.float32),
        pltpu.SemaphoreType.DMA,
        pltpu.SemaphoreType.DMA,
        pltpu.SemaphoreType.DMA,
        pltpu.SemaphoreType.DMA,
    ],
)


@jax.jit
def kernel(updates, mask):
  flat_updates = updates.reshape(-1)
  flat_mask = mask.reshape(-1).astype(jnp.int32)
  zeros = jnp.zeros((NS * ZSLICE,), jnp.float32)
  out = _scatter_add(flat_mask, flat_updates, zeros)
  return out.reshape(B, 2 * H, 2 * W_IN, C)
